# baseline (device time: 85879 ns/iter reference)
import jax
import jax.numpy as jnp
from jax import lax
from jax.experimental import pallas as pl
from jax.experimental.pallas import tpu as pltpu

N_DEV = 8
N_C = 1024
N_BUF = 2
PHASES = ((0, 1), (1, 3), (4, 4))


def kernel(x, w_mat):
    m_total, k_shard = x.shape
    k_total, n = w_mat.shape
    m_per = m_total // N_DEV
    n_chunks = n // N_C

    def body(x_ref, w_hbm, out_ref,
             xs_ref, xg_ref, w_buf, rhs_bf, send_sems, recv_sems, w_sems):
        my = lax.axis_index("i")

        barrier = pltpu.get_barrier_semaphore()
        for d in range(1, N_DEV):
            pl.semaphore_signal(
                barrier, inc=1,
                device_id=((my + d) % N_DEV,),
                device_id_type=pl.DeviceIdType.MESH,
            )
        pl.semaphore_wait(barrier, N_DEV - 1)

        xs_ref[...] = x_ref[...].astype(jnp.bfloat16)
        xg_ref[:, pl.ds(0, k_shard)] = xs_ref[pl.ds(my * m_per, m_per), :]

        def make_rdma(d):
            tgt = (my + d) % N_DEV
            return pltpu.make_async_remote_copy(
                src_ref=xs_ref.at[pl.ds(tgt * m_per, m_per), :],
                dst_ref=xg_ref.at[:, pl.ds(d * k_shard, k_shard)],
                send_sem=send_sems.at[d - 1],
                recv_sem=recv_sems.at[d - 1],
                device_id=(tgt,),
                device_id_type=pl.DeviceIdType.MESH,
            )

        rdmas = [make_rdma(d) for d in range(1, N_DEV)]
        for d in (1, 2, 3):
            rdmas[d - 1].start()

        jobs = [(p, c) for p in range(len(PHASES)) for c in range(n_chunks)]

        def fetch(job_idx):
            p, c = jobs[job_idx]
            jb0, nb = PHASES[p]
            slot = job_idx % N_BUF
            cps = []
            for i in range(nb):
                src_blk = (my + (N_DEV - (jb0 + i))) % N_DEV
                cp = pltpu.make_async_copy(
                    w_hbm.at[pl.ds(src_blk * k_shard, k_shard),
                             pl.ds(c * N_C, N_C)],
                    w_buf.at[slot, pl.ds(i * k_shard, k_shard), :],
                    w_sems.at[slot],
                )
                cp.start()
                cps.append(cp)
            return cps

        inflight = {}
        for j in range(N_BUF):
            inflight[j] = fetch(j)

        def convert(job_idx):
            p, _ = jobs[job_idx]
            kk = PHASES[p][1] * k_shard
            for cp in inflight.pop(job_idx):
                cp.wait()
            rhs_bf[job_idx % 2, pl.ds(0, kk), :] = w_buf[
                job_idx % N_BUF, pl.ds(0, kk), :
            ].astype(jnp.bfloat16)
            if job_idx + N_BUF < len(jobs):
                inflight[job_idx + N_BUF] = fetch(job_idx + N_BUF)

        convert(0)
        hi_started = False
        for job_idx, (p, c) in enumerate(jobs):
            jb0, nb = PHASES[p]
            kk = nb * k_shard
            if p == 1 and c == 0:
                for d in (1, 2, 3):
                    rdmas[d - 1].wait_recv()
            if p == 2 and c == 0:
                for d in (4, 5, 6, 7):
                    rdmas[d - 1].wait_recv()
            if job_idx + 1 < len(jobs):
                convert(job_idx + 1)
            acc = jnp.dot(
                xg_ref[:, pl.ds(jb0 * k_shard, kk)],
                rhs_bf[job_idx % 2, pl.ds(0, kk), :],
                preferred_element_type=jnp.float32,
            )
            cs = pl.ds(c * N_C, N_C)
            if p == 0:
                out_ref[:, cs] = acc.astype(jnp.bfloat16)
            elif p == 1:
                out_ref[:, cs] = (
                    out_ref[:, cs].astype(jnp.float32) + acc
                ).astype(jnp.bfloat16)
            else:
                out_ref[:, cs] = jnp.maximum(
                    out_ref[:, cs].astype(jnp.float32) + acc, 0.0
                ).astype(jnp.bfloat16)
            if p == 0 and c == 3 and not hi_started:
                for d in (1, 2, 3):
                    rdmas[d - 1].wait_send()
                for d in (4, 5, 6, 7):
                    rdmas[d - 1].start()
                hi_started = True

        for d in (4, 5, 6, 7):
            rdmas[d - 1].wait_send()

    return pl.pallas_call(
        body,
        out_shape=jax.ShapeDtypeStruct((m_per, n), jnp.bfloat16),
        in_specs=[
            pl.BlockSpec(memory_space=pltpu.VMEM),
            pl.BlockSpec(memory_space=pltpu.MemorySpace.HBM),
        ],
        out_specs=pl.BlockSpec(memory_space=pltpu.VMEM),
        scratch_shapes=[
            pltpu.VMEM((m_total, k_shard), jnp.bfloat16),
            pltpu.VMEM((m_per, k_total), jnp.bfloat16),
            pltpu.VMEM((N_BUF, 4 * k_shard, N_C), jnp.float32),
            pltpu.VMEM((2, 4 * k_shard, N_C), jnp.bfloat16),
            pltpu.SemaphoreType.DMA((N_DEV - 1,)),
            pltpu.SemaphoreType.DMA((N_DEV - 1,)),
            pltpu.SemaphoreType.DMA((N_BUF,)),
        ],
        compiler_params=pltpu.CompilerParams(
            collective_id=0,
            vmem_limit_bytes=100 * 1024 * 1024,
        ),
    )(x, w_mat)


# device time: 76625 ns/iter; 1.1208x vs baseline; 1.1208x over previous
import jax
import jax.numpy as jnp
from jax import lax
from jax.experimental import pallas as pl
from jax.experimental.pallas import tpu as pltpu

N_DEV = 8
N_C = 1024
N_BUF = 3
PHASES = ((0, 1), (1, 3), (4, 4))


def kernel(x, w_mat):
    m_total, k_shard = x.shape
    k_total, n = w_mat.shape
    m_per = m_total // N_DEV
    n_chunks = n // N_C

    def body(x_ref, w_hbm, out_ref,
             xs_ref, xg_ref, w_buf, send_sems, recv_sems, w_sems):
        my = lax.axis_index("i")

        barrier = pltpu.get_barrier_semaphore()
        for d in range(1, N_DEV):
            pl.semaphore_signal(
                barrier, inc=1,
                device_id=((my + d) % N_DEV,),
                device_id_type=pl.DeviceIdType.MESH,
            )
        pl.semaphore_wait(barrier, N_DEV - 1)

        xs_ref[...] = x_ref[...].astype(jnp.bfloat16)
        xg_ref[:, pl.ds(0, k_shard)] = xs_ref[pl.ds(my * m_per, m_per), :]

        def make_rdma(d):
            tgt = (my + d) % N_DEV
            return pltpu.make_async_remote_copy(
                src_ref=xs_ref.at[pl.ds(tgt * m_per, m_per), :],
                dst_ref=xg_ref.at[:, pl.ds(d * k_shard, k_shard)],
                send_sem=send_sems.at[d - 1],
                recv_sem=recv_sems.at[d - 1],
                device_id=(tgt,),
                device_id_type=pl.DeviceIdType.MESH,
            )

        rdmas = [make_rdma(d) for d in range(1, N_DEV)]
        for d in (1, 2, 3):
            rdmas[d - 1].start()

        jobs = [(p, c) for p in range(len(PHASES)) for c in range(n_chunks)]

        def fetch(job_idx):
            p, c = jobs[job_idx]
            jb0, nb = PHASES[p]
            slot = job_idx % N_BUF
            cps = []
            for i in range(nb):
                src_blk = (my + (N_DEV - (jb0 + i))) % N_DEV
                cp = pltpu.make_async_copy(
                    w_hbm.at[pl.ds(src_blk * k_shard, k_shard),
                             pl.ds(c * N_C, N_C)],
                    w_buf.at[slot, pl.ds(i * k_shard, k_shard), :],
                    w_sems.at[slot],
                )
                cp.start()
                cps.append(cp)
            return cps

        inflight = {}
        for j in range(N_BUF):
            inflight[j] = fetch(j)

        hi_started = False
        for job_idx, (p, c) in enumerate(jobs):
            jb0, nb = PHASES[p]
            kk = nb * k_shard
            if p == 1 and c == 0:
                for d in (1, 2, 3):
                    rdmas[d - 1].wait_recv()
            if p == 2 and c == 0:
                for d in (4, 5, 6, 7):
                    rdmas[d - 1].wait_recv()
            for cp in inflight.pop(job_idx):
                cp.wait()
            rhs = w_buf[job_idx % N_BUF, pl.ds(0, kk), :].astype(jnp.bfloat16)
            if job_idx + N_BUF < len(jobs):
                inflight[job_idx + N_BUF] = fetch(job_idx + N_BUF)
            acc = jnp.dot(
                xg_ref[:, pl.ds(jb0 * k_shard, kk)], rhs,
                preferred_element_type=jnp.float32,
            )
            cs = pl.ds(c * N_C, N_C)
            if p == 0:
                out_ref[:, cs] = acc.astype(jnp.bfloat16)
            elif p == 1:
                out_ref[:, cs] = (
                    out_ref[:, cs].astype(jnp.float32) + acc
                ).astype(jnp.bfloat16)
            else:
                out_ref[:, cs] = jnp.maximum(
                    out_ref[:, cs].astype(jnp.float32) + acc, 0.0
                ).astype(jnp.bfloat16)
            if p == 0 and c == 3 and not hi_started:
                for d in (1, 2, 3):
                    rdmas[d - 1].wait_send()
                for d in (4, 5, 6, 7):
                    rdmas[d - 1].start()
                hi_started = True

        for d in (4, 5, 6, 7):
            rdmas[d - 1].wait_send()

    return pl.pallas_call(
        body,
        out_shape=jax.ShapeDtypeStruct((m_per, n), jnp.bfloat16),
        in_specs=[
            pl.BlockSpec(memory_space=pltpu.VMEM),
            pl.BlockSpec(memory_space=pltpu.MemorySpace.HBM),
        ],
        out_specs=pl.BlockSpec(memory_space=pltpu.VMEM),
        scratch_shapes=[
            pltpu.VMEM((m_total, k_shard), jnp.bfloat16),
            pltpu.VMEM((m_per, k_total), jnp.bfloat16),
            pltpu.VMEM((N_BUF, 4 * k_shard, N_C), jnp.float32),
            pltpu.SemaphoreType.DMA((N_DEV - 1,)),
            pltpu.SemaphoreType.DMA((N_DEV - 1,)),
            pltpu.SemaphoreType.DMA((N_BUF,)),
        ],
        compiler_params=pltpu.CompilerParams(
            collective_id=0,
            vmem_limit_bytes=100 * 1024 * 1024,
        ),
    )(x, w_mat)
